# baseline (speedup 1.0000x reference)
import functools

import jax
import jax.numpy as jnp
from jax import lax
from jax.experimental import pallas as pl
from jax.experimental.pallas import tpu as pltpu
from jax.experimental.pallas import tpu_sc as plsc

N = 10000
D = 128
E = 320000
NC = 2
NS = 16
NW = NC * NS
EPW = E // NW
K = 50
NCHUNK = EPW // K
NBUF = 4
RPZ = 1000
ZT = N // RPZ
CW = 16

_mesh = plsc.VectorSubcoreMesh(core_axis_name="c", subcore_axis_name="s")


@functools.partial(
    pl.kernel,
    out_type=jax.ShapeDtypeStruct((NC, N, CW), jnp.float32),
    mesh=_mesh,
    compiler_params=pltpu.CompilerParams(use_tc_tiling_on_sc=False),
    scratch_types=[
        pltpu.VMEM((NCHUNK, K), jnp.int32),
        pltpu.VMEM((K, CW), jnp.float32),
        pltpu.VMEM((RPZ, CW), jnp.float32),
        pltpu.VMEM_SHARED((N, CW), jnp.float32),
        pltpu.SemaphoreType.DMA,
    ],
)
def _count_kernel(ei_hbm, ones_hbm, out_hbm, dst_v, ones_v, zb_v, acc_sh, csem):
    c = lax.axis_index("c")
    s = lax.axis_index("s")
    wid = s * NC + c
    pltpu.sync_copy(ei_hbm.at[1, wid], dst_v)
    pltpu.sync_copy(ones_hbm, ones_v)

    def _zrow(r, carry):
        zb_v[r, :] = jnp.zeros((CW,), jnp.float32)
        return carry

    lax.fori_loop(0, RPZ, _zrow, 0)
    row0 = s * RPZ

    @pl.when(s < ZT)
    def _zero():
        pltpu.sync_copy(zb_v, acc_sh.at[pl.ds(row0, RPZ)])

    plsc.subcore_barrier()

    def _chunk(j, carry):
        pltpu.async_copy(ones_v, acc_sh.at[dst_v.at[j]], csem, add=True)
        return carry

    lax.fori_loop(0, NCHUNK, _chunk, 0)

    def _drain(j, carry):
        pltpu.make_async_copy(ones_v, acc_sh.at[dst_v.at[j]], csem).wait()
        return carry

    lax.fori_loop(0, NCHUNK, _drain, 0)
    plsc.subcore_barrier()

    @pl.when(s < ZT)
    def _dump():
        pltpu.sync_copy(acc_sh.at[pl.ds(row0, RPZ)], out_hbm.at[c, pl.ds(row0, RPZ)])


@functools.partial(
    pl.kernel,
    out_type=jax.ShapeDtypeStruct((NC, N, D), jnp.float32),
    mesh=_mesh,
    compiler_params=pltpu.CompilerParams(use_tc_tiling_on_sc=False),
    scratch_types=[
        pltpu.VMEM((NCHUNK, K), jnp.int32),
        pltpu.VMEM((NCHUNK, K), jnp.int32),
        pltpu.VMEM((NBUF, K, D), jnp.float32),
        pltpu.VMEM_SHARED((N, D), jnp.float32),
        [pltpu.SemaphoreType.DMA] * NBUF,
        [pltpu.SemaphoreType.DMA] * NBUF,
    ],
)
def _agg_kernel(table_hbm, ei_hbm, out_hbm, src_v, dst_v, bufs, acc_sh, semG, semS):
    c = lax.axis_index("c")
    s = lax.axis_index("s")
    wid = s * NC + c
    pltpu.sync_copy(ei_hbm.at[0, wid], src_v)
    pltpu.sync_copy(ei_hbm.at[1, wid], dst_v)

    zb = bufs.at[0]

    def _zrow(r, carry):
        def _zcol(i, carry2):
            bufs[0, r, pl.ds(i * 16, 16)] = jnp.zeros((16,), jnp.float32)
            return carry2

        return lax.fori_loop(0, D // 16, _zcol, carry)

    lax.fori_loop(0, K, _zrow, 0)
    row0 = s * RPZ

    @pl.when(s < ZT)
    def _zero():
        for t in range(RPZ // K):
            pltpu.sync_copy(zb, acc_sh.at[pl.ds(row0 + t * K, K)])

    plsc.subcore_barrier()

    for b in range(NBUF):
        pltpu.async_copy(table_hbm.at[src_v.at[b]], bufs.at[b], semG[b])

    def _round(t, carry):
        j0 = NBUF * t
        for b in range(NBUF):
            jj = j0 + b
            pltpu.make_async_copy(
                table_hbm.at[src_v.at[jj]], bufs.at[b], semG[b]
            ).wait()
            pltpu.async_copy(bufs.at[b], acc_sh.at[dst_v.at[jj]], semS[b], add=True)
        for b in range(NBUF):
            jj = j0 + b
            jn = jj + NBUF
            pltpu.make_async_copy(
                bufs.at[b], acc_sh.at[dst_v.at[jj]], semS[b]
            ).wait()

            @pl.when(jn < NCHUNK)
            def _next():
                pltpu.async_copy(table_hbm.at[src_v.at[jn]], bufs.at[b], semG[b])

        return carry

    lax.fori_loop(0, NCHUNK // NBUF, _round, 0)
    plsc.subcore_barrier()

    @pl.when(s < ZT)
    def _dump():
        pltpu.sync_copy(acc_sh.at[pl.ds(row0, RPZ)], out_hbm.at[c, pl.ds(row0, RPZ)])


def _mm_body(x_ref, w_ref, o_ref):
    o_ref[...] = jnp.dot(x_ref[...], w_ref[...], preferred_element_type=jnp.float32)


def _mm_body(x_ref, w_ref, o_ref):
    o_ref[...] = jnp.dot(x_ref[...], w_ref[...], preferred_element_type=jnp.float32)


def _prescale_body(xw_ref, cnt_ref, xws_ref, dis_ref, ic_ref):
    ce = cnt_ref[0, :, 0:1] + cnt_ref[1, :, 0:1]
    dis = lax.rsqrt(ce + 1.0)
    xws_ref[...] = dis * xw_ref[...]
    dis_ref[...] = dis
    ic_ref[...] = 1.0 / jnp.maximum(ce, 1.0)


def _gcn_finish_body(s_ref, xws_ref, dis_ref, b_ref, h_ref):
    tot = s_ref[0] + s_ref[1] + xws_ref[...]
    h_ref[...] = jnp.maximum(dis_ref[...] * tot + b_ref[...], 0.0)


def _sage_body(t_ref, h_ref, ic_ref, wl_ref, bl_ref, wr_ref, out_ref):
    mean = (t_ref[0] + t_ref[1]) * ic_ref[...]
    out = (
        jnp.dot(mean, wl_ref[...], preferred_element_type=jnp.float32)
        + bl_ref[...]
        + jnp.dot(h_ref[...], wr_ref[...], preferred_element_type=jnp.float32)
    )
    out_ref[...] = jnp.maximum(out, 0.0)


def kernel(feat, ei, W_gcn, b_gcn, W_l, b_l, W_r):
    eir = ei.astype(jnp.int32).reshape(2, NW, NCHUNK, K)
    ones = jnp.ones((K, CW), jnp.float32)

    xw = pl.pallas_call(
        _mm_body,
        out_shape=jax.ShapeDtypeStruct((N, D), jnp.float32),
    )(feat, W_gcn)

    cnt_parts = _count_kernel(eir, ones)

    xws, dis, ic = pl.pallas_call(
        _prescale_body,
        out_shape=[
            jax.ShapeDtypeStruct((N, D), jnp.float32),
            jax.ShapeDtypeStruct((N, 1), jnp.float32),
            jax.ShapeDtypeStruct((N, 1), jnp.float32),
        ],
    )(xw, cnt_parts)

    S = _agg_kernel(xws, eir)

    h = pl.pallas_call(
        _gcn_finish_body,
        out_shape=jax.ShapeDtypeStruct((N, D), jnp.float32),
    )(S, xws, dis, jnp.reshape(b_gcn, (1, D)))

    T = _agg_kernel(h, eir)

    out = pl.pallas_call(
        _sage_body,
        out_shape=jax.ShapeDtypeStruct((N, D), jnp.float32),
    )(T, h, ic, W_l, jnp.reshape(b_l, (1, D)), W_r)
    return out


# speedup vs baseline: 1.0077x; 1.0077x over previous
import functools

import jax
import jax.numpy as jnp
from jax import lax
from jax.experimental import pallas as pl
from jax.experimental.pallas import tpu as pltpu
from jax.experimental.pallas import tpu_sc as plsc

N = 10000
D = 128
E = 320000
NC = 2
NS = 16
NW = NC * NS
EPW = E // NW
K = 50
NCHUNK = EPW // K
NBUF = 4
RPZ = 1000
ZT = N // RPZ
CW = 16

_mesh = plsc.VectorSubcoreMesh(core_axis_name="c", subcore_axis_name="s")


@functools.partial(
    pl.kernel,
    out_type=jax.ShapeDtypeStruct((NC, N, CW), jnp.float32),
    mesh=_mesh,
    compiler_params=pltpu.CompilerParams(use_tc_tiling_on_sc=False),
    scratch_types=[
        pltpu.VMEM((NCHUNK, K), jnp.int32),
        pltpu.VMEM((K, CW), jnp.float32),
        pltpu.VMEM((RPZ, CW), jnp.float32),
        pltpu.VMEM_SHARED((N, CW), jnp.float32),
        pltpu.SemaphoreType.DMA,
    ],
)
def _count_kernel(dst_hbm, ones_hbm, out_hbm, dst_v, ones_v, zb_v, acc_sh, csem):
    c = lax.axis_index("c")
    s = lax.axis_index("s")
    wid = s * NC + c
    pltpu.sync_copy(dst_hbm.at[wid], dst_v)
    pltpu.sync_copy(ones_hbm, ones_v)

    def _zrow(r, carry):
        zb_v[r, :] = jnp.zeros((CW,), jnp.float32)
        return carry

    lax.fori_loop(0, RPZ, _zrow, 0)
    row0 = s * RPZ

    @pl.when(s < ZT)
    def _zero():
        pltpu.sync_copy(zb_v, acc_sh.at[pl.ds(row0, RPZ)])

    plsc.subcore_barrier()

    def _chunk(j, carry):
        pltpu.async_copy(ones_v, acc_sh.at[dst_v.at[j]], csem, add=True)
        return carry

    lax.fori_loop(0, NCHUNK, _chunk, 0)

    def _drain(j, carry):
        pltpu.make_async_copy(ones_v, acc_sh.at[dst_v.at[j]], csem).wait()
        return carry

    lax.fori_loop(0, NCHUNK, _drain, 0)
    plsc.subcore_barrier()

    @pl.when(s < ZT)
    def _dump():
        pltpu.sync_copy(acc_sh.at[pl.ds(row0, RPZ)], out_hbm.at[c, pl.ds(row0, RPZ)])


@functools.partial(
    pl.kernel,
    out_type=jax.ShapeDtypeStruct((NC, N, D), jnp.float32),
    mesh=_mesh,
    compiler_params=pltpu.CompilerParams(use_tc_tiling_on_sc=False),
    scratch_types=[
        pltpu.VMEM((NCHUNK, K), jnp.int32),
        pltpu.VMEM((NCHUNK, K), jnp.int32),
        pltpu.VMEM((NBUF, K, D), jnp.float32),
        pltpu.VMEM_SHARED((N, D), jnp.float32),
        [pltpu.SemaphoreType.DMA] * NBUF,
        [pltpu.SemaphoreType.DMA] * NBUF,
    ],
)
def _agg_kernel(
    table_hbm, src_hbm, dstr_hbm, out_hbm, src_v, dst_v, bufs, acc_sh, semG, semS
):
    c = lax.axis_index("c")
    s = lax.axis_index("s")
    wid = s * NC + c
    pltpu.sync_copy(src_hbm.at[wid], src_v)
    pltpu.sync_copy(dstr_hbm.at[wid], dst_v)

    zb = bufs.at[0]

    def _zrow(r, carry):
        def _zcol(i, carry2):
            bufs[0, r, pl.ds(i * 16, 16)] = jnp.zeros((16,), jnp.float32)
            return carry2

        return lax.fori_loop(0, D // 16, _zcol, carry)

    lax.fori_loop(0, K, _zrow, 0)
    row0 = s * RPZ

    @pl.when(s < ZT)
    def _zero():
        for t in range(RPZ // K):
            pltpu.sync_copy(zb, acc_sh.at[pl.ds(row0 + t * K, K)])

    plsc.subcore_barrier()

    for b in range(NBUF):
        pltpu.async_copy(table_hbm.at[src_v.at[b]], bufs.at[b], semG[b])

    def _round(t, carry):
        j0 = NBUF * t
        for b in range(NBUF):
            jj = j0 + b
            pltpu.make_async_copy(
                table_hbm.at[src_v.at[jj]], bufs.at[b], semG[b]
            ).wait()
            pltpu.async_copy(bufs.at[b], acc_sh.at[dst_v.at[jj]], semS[b], add=True)
        for b in range(NBUF):
            jj = j0 + b
            jn = jj + NBUF
            pltpu.make_async_copy(
                bufs.at[b], acc_sh.at[dst_v.at[jj]], semS[b]
            ).wait()

            @pl.when(jn < NCHUNK)
            def _next():
                pltpu.async_copy(table_hbm.at[src_v.at[jn]], bufs.at[b], semG[b])

        return carry

    lax.fori_loop(0, NCHUNK // NBUF, _round, 0)
    plsc.subcore_barrier()

    @pl.when(s < ZT)
    def _dump():
        pltpu.sync_copy(acc_sh.at[pl.ds(row0, RPZ)], out_hbm.at[c, pl.ds(row0, RPZ)])


def _mm_body(x_ref, w_ref, o_ref):
    o_ref[...] = jnp.dot(x_ref[...], w_ref[...], preferred_element_type=jnp.float32)


def _mm_body(x_ref, w_ref, o_ref):
    o_ref[...] = jnp.dot(x_ref[...], w_ref[...], preferred_element_type=jnp.float32)


def _prescale_body(xw_ref, cnt_ref, xws_ref, dis_ref, ic_ref):
    ce = cnt_ref[0, :, 0:1] + cnt_ref[1, :, 0:1]
    dis = lax.rsqrt(ce + 1.0)
    xws_ref[...] = dis * xw_ref[...]
    dis_ref[...] = dis
    ic_ref[...] = 1.0 / jnp.maximum(ce, 1.0)


def _gcn_finish_body(s_ref, xws_ref, dis_ref, b_ref, h_ref):
    tot = s_ref[0] + s_ref[1] + xws_ref[...]
    h_ref[...] = jnp.maximum(dis_ref[...] * tot + b_ref[...], 0.0)


def _sage_body(t_ref, h_ref, ic_ref, wl_ref, bl_ref, wr_ref, out_ref):
    mean = (t_ref[0] + t_ref[1]) * ic_ref[...]
    out = (
        jnp.dot(mean, wl_ref[...], preferred_element_type=jnp.float32)
        + bl_ref[...]
        + jnp.dot(h_ref[...], wr_ref[...], preferred_element_type=jnp.float32)
    )
    out_ref[...] = jnp.maximum(out, 0.0)


def kernel(feat, ei, W_gcn, b_gcn, W_l, b_l, W_r):
    src = ei[0].astype(jnp.int32).reshape(NW, NCHUNK, K)
    dst = ei[1].astype(jnp.int32).reshape(NW, NCHUNK, K)
    ones = jnp.ones((K, CW), jnp.float32)

    xw = pl.pallas_call(
        _mm_body,
        out_shape=jax.ShapeDtypeStruct((N, D), jnp.float32),
    )(feat, W_gcn)

    cnt_parts = _count_kernel(dst, ones)

    xws, dis, ic = pl.pallas_call(
        _prescale_body,
        out_shape=[
            jax.ShapeDtypeStruct((N, D), jnp.float32),
            jax.ShapeDtypeStruct((N, 1), jnp.float32),
            jax.ShapeDtypeStruct((N, 1), jnp.float32),
        ],
    )(xw, cnt_parts)

    S = _agg_kernel(xws, src, dst)

    h = pl.pallas_call(
        _gcn_finish_body,
        out_shape=jax.ShapeDtypeStruct((N, D), jnp.float32),
    )(S, xws, dis, jnp.reshape(b_gcn, (1, D)))

    T = _agg_kernel(h, src, dst)

    out = pl.pallas_call(
        _sage_body,
        out_shape=jax.ShapeDtypeStruct((N, D), jnp.float32),
    )(T, h, ic, W_l, jnp.reshape(b_l, (1, D)), W_r)
    return out


# speedup vs baseline: 1.0083x; 1.0006x over previous
import functools

import jax
import jax.numpy as jnp
from jax import lax
from jax.experimental import pallas as pl
from jax.experimental.pallas import tpu as pltpu
from jax.experimental.pallas import tpu_sc as plsc

N = 10000
D = 128
E = 320000
NC = 2
NS = 16
NW = NC * NS
EPW = E // NW
K = 50
NCHUNK = EPW // K
NBUF = 4
RPZ = 1000
ZT = N // RPZ
CW = 16

_mesh = plsc.VectorSubcoreMesh(core_axis_name="c", subcore_axis_name="s")


@functools.partial(
    pl.kernel,
    out_type=jax.ShapeDtypeStruct((NC, N, CW), jnp.float32),
    mesh=_mesh,
    compiler_params=pltpu.CompilerParams(use_tc_tiling_on_sc=False),
    scratch_types=[
        pltpu.VMEM((NCHUNK, K), jnp.int32),
        pltpu.VMEM((K, CW), jnp.float32),
        pltpu.VMEM((RPZ, CW), jnp.float32),
        pltpu.VMEM_SHARED((N, CW), jnp.float32),
        pltpu.SemaphoreType.DMA,
    ],
)
def _count_kernel(dst_hbm, ones_hbm, out_hbm, dst_v, ones_v, zb_v, acc_sh, csem):
    c = lax.axis_index("c")
    s = lax.axis_index("s")
    wid = s * NC + c
    pltpu.sync_copy(dst_hbm.at[wid], dst_v)
    pltpu.sync_copy(ones_hbm, ones_v)

    def _zrow(r, carry):
        zb_v[r, :] = jnp.zeros((CW,), jnp.float32)
        return carry

    lax.fori_loop(0, RPZ, _zrow, 0)
    row0 = s * RPZ

    @pl.when(s < ZT)
    def _zero():
        pltpu.sync_copy(zb_v, acc_sh.at[pl.ds(row0, RPZ)])

    plsc.subcore_barrier()

    def _chunk(j, carry):
        pltpu.async_copy(ones_v, acc_sh.at[dst_v.at[j]], csem, add=True)
        return carry

    lax.fori_loop(0, NCHUNK, _chunk, 0)

    def _drain(j, carry):
        pltpu.make_async_copy(ones_v, acc_sh.at[dst_v.at[j]], csem).wait()
        return carry

    lax.fori_loop(0, NCHUNK, _drain, 0)
    plsc.subcore_barrier()

    @pl.when(s < ZT)
    def _dump():
        pltpu.sync_copy(acc_sh.at[pl.ds(row0, RPZ)], out_hbm.at[c, pl.ds(row0, RPZ)])


@functools.partial(
    pl.kernel,
    out_type=jax.ShapeDtypeStruct((NC, N, D), jnp.float32),
    mesh=_mesh,
    compiler_params=pltpu.CompilerParams(use_tc_tiling_on_sc=False),
    scratch_types=[
        pltpu.VMEM((NCHUNK, K), jnp.int32),
        pltpu.VMEM((NCHUNK, K), jnp.int32),
        pltpu.VMEM((NBUF, K, D), jnp.float32),
        pltpu.VMEM_SHARED((N, D), jnp.float32),
        [pltpu.SemaphoreType.DMA] * NBUF,
        [pltpu.SemaphoreType.DMA] * NBUF,
    ],
)
def _agg_kernel(
    table_hbm, src_hbm, dstr_hbm, out_hbm, src_v, dst_v, bufs, acc_sh, semG, semS
):
    c = lax.axis_index("c")
    s = lax.axis_index("s")
    wid = s * NC + c
    pltpu.sync_copy(src_hbm.at[wid], src_v)
    pltpu.sync_copy(dstr_hbm.at[wid], dst_v)

    zb = bufs.at[0]

    def _zrow(r, carry):
        def _zcol(i, carry2):
            bufs[0, r, pl.ds(i * 16, 16)] = jnp.zeros((16,), jnp.float32)
            return carry2

        return lax.fori_loop(0, D // 16, _zcol, carry)

    lax.fori_loop(0, K, _zrow, 0)
    row0 = s * RPZ

    @pl.when(s < ZT)
    def _zero():
        for t in range(RPZ // K):
            pltpu.sync_copy(zb, acc_sh.at[pl.ds(row0 + t * K, K)])

    plsc.subcore_barrier()

    for b in range(NBUF):
        pltpu.async_copy(table_hbm.at[src_v.at[b]], bufs.at[b], semG[b])

    def _round(t, carry):
        j0 = NBUF * t
        for b in range(NBUF):
            jj = j0 + b
            pltpu.make_async_copy(
                table_hbm.at[src_v.at[jj]], bufs.at[b], semG[b]
            ).wait()
            pltpu.async_copy(bufs.at[b], acc_sh.at[dst_v.at[jj]], semS[b], add=True)
        for b in range(NBUF):
            jj = j0 + b
            jn = jj + NBUF
            pltpu.make_async_copy(
                bufs.at[b], acc_sh.at[dst_v.at[jj]], semS[b]
            ).wait()

            @pl.when(jn < NCHUNK)
            def _next():
                pltpu.async_copy(table_hbm.at[src_v.at[jn]], bufs.at[b], semG[b])

        return carry

    lax.fori_loop(0, NCHUNK // NBUF, _round, 0)
    plsc.subcore_barrier()

    @pl.when(s < ZT)
    def _dump():
        pltpu.sync_copy(acc_sh.at[pl.ds(row0, RPZ)], out_hbm.at[c, pl.ds(row0, RPZ)])


def _mm_body(x_ref, w_ref, o_ref):
    o_ref[...] = jnp.dot(x_ref[...], w_ref[...], preferred_element_type=jnp.float32)


def _prescale_body(xw_ref, cnt_ref, xws_ref, dis_ref, ic_ref):
    ce = cnt_ref[0, :, 0:1] + cnt_ref[1, :, 0:1]
    dis = lax.rsqrt(ce + 1.0)
    xws_ref[...] = dis * xw_ref[...]
    dis_ref[...] = dis
    ic_ref[...] = 1.0 / jnp.maximum(ce, 1.0)


def _gcn_finish_body(s_ref, xws_ref, dis_ref, b_ref, h_ref):
    tot = s_ref[0] + s_ref[1] + xws_ref[...]
    h_ref[...] = jnp.maximum(dis_ref[...] * tot + b_ref[...], 0.0)


def _sage_body(t_ref, h_ref, ic_ref, wl_ref, bl_ref, wr_ref, out_ref):
    mean = (t_ref[0] + t_ref[1]) * ic_ref[...]
    out = (
        jnp.dot(mean, wl_ref[...], preferred_element_type=jnp.float32)
        + bl_ref[...]
        + jnp.dot(h_ref[...], wr_ref[...], preferred_element_type=jnp.float32)
    )
    out_ref[...] = jnp.maximum(out, 0.0)


def kernel(feat, ei, W_gcn, b_gcn, W_l, b_l, W_r):
    src = ei[0].astype(jnp.int32).reshape(NW, NCHUNK, K)
    dst = ei[1].astype(jnp.int32).reshape(NW, NCHUNK, K)
    ones = jnp.ones((K, CW), jnp.float32)

    xw = pl.pallas_call(
        _mm_body,
        out_shape=jax.ShapeDtypeStruct((N, D), jnp.float32),
    )(feat, W_gcn)

    cnt_parts = _count_kernel(dst, ones)

    xws, dis, ic = pl.pallas_call(
        _prescale_body,
        out_shape=[
            jax.ShapeDtypeStruct((N, D), jnp.float32),
            jax.ShapeDtypeStruct((N, 1), jnp.float32),
            jax.ShapeDtypeStruct((N, 1), jnp.float32),
        ],
    )(xw, cnt_parts)

    S = _agg_kernel(xws, src, dst)

    h = pl.pallas_call(
        _gcn_finish_body,
        out_shape=jax.ShapeDtypeStruct((N, D), jnp.float32),
    )(S, xws, dis, jnp.reshape(b_gcn, (1, D)))

    T = _agg_kernel(h, src, dst)

    out = pl.pallas_call(
        _sage_body,
        out_shape=jax.ShapeDtypeStruct((N, D), jnp.float32),
    )(T, h, ic, W_l, jnp.reshape(b_l, (1, D)), W_r)
    return out
